# baseline (speedup 1.0000x reference)
import functools

import jax
import jax.numpy as jnp
from jax import lax
from jax.experimental import pallas as pl
from jax.experimental.pallas import tpu as pltpu
from jax.experimental.pallas import tpu_sc as plsc

VOCAB = 30522
EXTRA = 2
DIM = 768
HDIM = DIM // 2
LN_EPS = 1e-12

BATCH = 128
SEQ = 512
B = BATCH * SEQ

NC = 2
NS = 16
NW = NC * NS
NB = 128

KCH = 8
BC = B // KCH

_sc_mesh = plsc.VectorSubcoreMesh(core_axis_name="c", subcore_axis_name="s")

CAST_ROWS = 1024
VTOT = VOCAB + EXTRA
_NCAST = pl.cdiv(VTOT, CAST_ROWS)
_E_BASE = VOCAB - (_NCAST - 1) * CAST_ROWS


def _pack_pair(x):
    lo = x[:, :HDIM].astype(jnp.bfloat16).astype(jnp.float32)
    hi = x[:, HDIM:].astype(jnp.bfloat16).astype(jnp.float32)
    lob = lax.bitcast_convert_type(lo, jnp.int32)
    hib = lax.bitcast_convert_type(hi, jnp.int32)
    return jnp.bitwise_or(hib, lax.shift_right_logical(lob, 16))


def _cast_body(x_ref, e_ref, o_ref):
    i = pl.program_id(0)
    packed = _pack_pair(x_ref[...])

    @pl.when(i == _NCAST - 1)
    def _():
        pe = _pack_pair(e_ref[...])
        row = lax.broadcasted_iota(jnp.int32, (CAST_ROWS, 1), 0)
        p = jnp.where(row == _E_BASE, pe[0:1, :], packed)
        o_ref[...] = jnp.where(row == _E_BASE + 1, pe[1:2, :], p)

    @pl.when(i != _NCAST - 1)
    def _():
        o_ref[...] = packed


def _cast_table(word_emb, extra_emb):
    return pl.pallas_call(
        _cast_body,
        grid=(_NCAST,),
        in_specs=[
            pl.BlockSpec((CAST_ROWS, DIM), lambda i: (i, 0)),
            pl.BlockSpec((EXTRA, DIM), lambda i: (0, 0)),
        ],
        out_specs=pl.BlockSpec((CAST_ROWS, HDIM), lambda i: (i, 0)),
        out_shape=jax.ShapeDtypeStruct((VTOT, HDIM), jnp.int32),
    )(word_emb, extra_emb)


def _make_sc_gather(bt):
    bpw = bt // NW
    nchunk = bpw // NB

    @functools.partial(
        pl.kernel,
        mesh=_sc_mesh,
        out_type=jax.ShapeDtypeStruct((bt, HDIM), jnp.int32),
        scratch_types=[
            pltpu.VMEM((bpw,), jnp.int32),
            pltpu.VMEM((NB, HDIM), jnp.int32),
            pltpu.VMEM((NB, HDIM), jnp.int32),
            pltpu.SemaphoreType.DMA,
            pltpu.SemaphoreType.DMA,
            pltpu.SemaphoreType.DMA,
            pltpu.SemaphoreType.DMA,
        ],
    )
    def sc_gather(ids_hbm, table_hbm, out_hbm, idx_v, rows0, rows1, g0, g1, s0, s1):
        wid = lax.axis_index("s") * NC + lax.axis_index("c")
        base = wid * bpw
        pltpu.sync_copy(ids_hbm.at[pl.ds(base, bpw)], idx_v)
        rows = (rows0, rows1)
        gsem = (g0, g1)
        ssem = (s0, s1)

        def gather_start(c, b):
            pltpu.async_copy(table_hbm.at[idx_v.at[pl.ds(c * NB, NB)]], rows[b], gsem[b])

        def gather_wait(c, b):
            pltpu.make_async_copy(
                table_hbm.at[idx_v.at[pl.ds(c * NB, NB)]], rows[b], gsem[b]
            ).wait()

        def store_start(c, b):
            pltpu.async_copy(rows[b], out_hbm.at[pl.ds(base + c * NB, NB)], ssem[b])

        def store_wait(c, b):
            pltpu.make_async_copy(
                rows[b], out_hbm.at[pl.ds(base + c * NB, NB)], ssem[b]
            ).wait()

        gather_start(0, 0)

        def body(i, carry):
            c0 = i * 2
            for b in range(2):
                c = c0 + b
                gather_wait(c, b)

                @pl.when(c >= 1)
                def _():
                    store_wait(c - 1, 1 - b)

                @pl.when(c + 1 < nchunk)
                def _():
                    gather_start(c + 1, 1 - b)

                store_start(c, b)
            return carry

        lax.fori_loop(0, nchunk // 2, body, 0)
        store_wait(nchunk - 1, (nchunk - 1) % 2)

    return sc_gather


_sc_gather_chunk = _make_sc_gather(BC)


ROWS = 1024
NSEQ = ROWS // SEQ
GRID_C = BC // ROWS
_FINV = 1.0 / DIM


def _tc_tail(g_ref, pos_ref, gam_ref, bet_ref, out_ref):
    u = g_ref[...].reshape(NSEQ, SEQ, HDIM)
    lo = lax.bitcast_convert_type(lax.shift_left(u, 16), jnp.float32)
    hi = lax.bitcast_convert_type(
        jnp.bitwise_and(u, jnp.int32(-65536)), jnp.float32)
    xl = lo + pos_ref[:, :HDIM][None]
    xh = hi + pos_ref[:, HDIM:][None]
    s = (jnp.sum(xl, axis=2, keepdims=True) + jnp.sum(xh, axis=2, keepdims=True))
    q = (jnp.sum(xl * xl, axis=2, keepdims=True)
         + jnp.sum(xh * xh, axis=2, keepdims=True))
    mean = s * _FINV
    var = q * _FINV - mean * mean
    r = lax.rsqrt(var + LN_EPS)
    yl = (xl - mean) * r * gam_ref[:, :HDIM][None] + bet_ref[:, :HDIM][None]
    yh = (xh - mean) * r * gam_ref[:, HDIM:][None] + bet_ref[:, HDIM:][None]
    out_ref[:, :HDIM] = yl.reshape(ROWS, HDIM)
    out_ref[:, HDIM:] = yh.reshape(ROWS, HDIM)


def _tc_tail_alias(prev_ref, g_ref, pos_ref, gam_ref, bet_ref, out_ref):
    del prev_ref
    _tc_tail(g_ref, pos_ref, gam_ref, bet_ref, out_ref)


def _chunk_specs():
    return [
        pl.BlockSpec((ROWS, HDIM), lambda i: (i, 0)),
        pl.BlockSpec((SEQ, DIM), lambda i: (0, 0)),
        pl.BlockSpec((1, DIM), lambda i: (0, 0)),
        pl.BlockSpec((1, DIM), lambda i: (0, 0)),
    ]


def _tc_chunk_call(k, prev, g_k, pos_c, gam, bet):
    out_map = functools.partial(lambda kk, i: (i + kk * GRID_C, 0), k)
    out_shape = jax.ShapeDtypeStruct((B, DIM), jnp.float32)
    if prev is None:
        return pl.pallas_call(
            _tc_tail,
            grid=(GRID_C,),
            in_specs=_chunk_specs(),
            out_specs=pl.BlockSpec((ROWS, DIM), out_map),
            out_shape=out_shape,
        )(g_k, pos_c, gam, bet)
    return pl.pallas_call(
        _tc_tail_alias,
        grid=(GRID_C,),
        in_specs=[pl.BlockSpec(memory_space=pl.ANY)] + _chunk_specs(),
        out_specs=pl.BlockSpec((ROWS, DIM), out_map),
        out_shape=out_shape,
        input_output_aliases={0: 0},
    )(prev, g_k, pos_c, gam, bet)


def kernel(input_ids, word_emb, extra_emb, token_type_emb, pos_emb, ln_gamma, ln_beta):
    ids = input_ids.reshape(-1).astype(jnp.int32)
    pos_c = pos_emb + token_type_emb[0][None, :]
    gam = ln_gamma.reshape(1, DIM)
    bet = ln_beta.reshape(1, DIM)

    table = _cast_table(word_emb, extra_emb)
    gathered = [
        _sc_gather_chunk(lax.slice(ids, (k * BC,), ((k + 1) * BC,)), table)
        for k in range(KCH)
    ]
    out = None
    for k in range(KCH):
        out = _tc_chunk_call(k, out, gathered[k], pos_c, gam, bet)
    return out.reshape(BATCH, SEQ, DIM)


# speedup vs baseline: 1.0598x; 1.0598x over previous
import functools

import jax
import jax.numpy as jnp
from jax import lax
from jax.experimental import pallas as pl
from jax.experimental.pallas import tpu as pltpu
from jax.experimental.pallas import tpu_sc as plsc

VOCAB = 30522
EXTRA = 2
DIM = 768
HDIM = DIM // 2
LN_EPS = 1e-12

BATCH = 128
SEQ = 512
B = BATCH * SEQ

NC = 2
NS = 16
NW = NC * NS
NB = 128

KCH = 4
BC = B // KCH

_sc_mesh = plsc.VectorSubcoreMesh(core_axis_name="c", subcore_axis_name="s")

CAST_ROWS = 1024
VTOT = VOCAB + EXTRA
_NCAST = pl.cdiv(VTOT, CAST_ROWS)
_E_BASE = VOCAB - (_NCAST - 1) * CAST_ROWS


def _pack_pair(x):
    lo = x[:, :HDIM].astype(jnp.bfloat16).astype(jnp.float32)
    hi = x[:, HDIM:].astype(jnp.bfloat16).astype(jnp.float32)
    lob = lax.bitcast_convert_type(lo, jnp.int32)
    hib = lax.bitcast_convert_type(hi, jnp.int32)
    return jnp.bitwise_or(hib, lax.shift_right_logical(lob, 16))


def _cast_body(x_ref, e_ref, o_ref):
    i = pl.program_id(0)
    packed = _pack_pair(x_ref[...])

    @pl.when(i == _NCAST - 1)
    def _():
        pe = _pack_pair(e_ref[...])
        row = lax.broadcasted_iota(jnp.int32, (CAST_ROWS, 1), 0)
        p = jnp.where(row == _E_BASE, pe[0:1, :], packed)
        o_ref[...] = jnp.where(row == _E_BASE + 1, pe[1:2, :], p)

    @pl.when(i != _NCAST - 1)
    def _():
        o_ref[...] = packed


def _cast_table(word_emb, extra_emb):
    return pl.pallas_call(
        _cast_body,
        grid=(_NCAST,),
        in_specs=[
            pl.BlockSpec((CAST_ROWS, DIM), lambda i: (i, 0)),
            pl.BlockSpec((EXTRA, DIM), lambda i: (0, 0)),
        ],
        out_specs=pl.BlockSpec((CAST_ROWS, HDIM), lambda i: (i, 0)),
        out_shape=jax.ShapeDtypeStruct((VTOT, HDIM), jnp.int32),
    )(word_emb, extra_emb)


def _make_sc_gather(bt):
    bpw = bt // NW
    nchunk = bpw // NB

    @functools.partial(
        pl.kernel,
        mesh=_sc_mesh,
        out_type=jax.ShapeDtypeStruct((bt, HDIM), jnp.int32),
        scratch_types=[
            pltpu.VMEM((bpw,), jnp.int32),
            pltpu.VMEM((NB, HDIM), jnp.int32),
            pltpu.VMEM((NB, HDIM), jnp.int32),
            pltpu.SemaphoreType.DMA,
            pltpu.SemaphoreType.DMA,
            pltpu.SemaphoreType.DMA,
            pltpu.SemaphoreType.DMA,
        ],
    )
    def sc_gather(ids_hbm, table_hbm, out_hbm, idx_v, rows0, rows1, g0, g1, s0, s1):
        wid = lax.axis_index("s") * NC + lax.axis_index("c")
        base = wid * bpw
        pltpu.sync_copy(ids_hbm.at[pl.ds(base, bpw)], idx_v)
        rows = (rows0, rows1)
        gsem = (g0, g1)
        ssem = (s0, s1)

        def gather_start(c, b):
            pltpu.async_copy(table_hbm.at[idx_v.at[pl.ds(c * NB, NB)]], rows[b], gsem[b])

        def gather_wait(c, b):
            pltpu.make_async_copy(
                table_hbm.at[idx_v.at[pl.ds(c * NB, NB)]], rows[b], gsem[b]
            ).wait()

        def store_start(c, b):
            pltpu.async_copy(rows[b], out_hbm.at[pl.ds(base + c * NB, NB)], ssem[b])

        def store_wait(c, b):
            pltpu.make_async_copy(
                rows[b], out_hbm.at[pl.ds(base + c * NB, NB)], ssem[b]
            ).wait()

        gather_start(0, 0)

        def body(i, carry):
            c0 = i * 2
            for b in range(2):
                c = c0 + b
                gather_wait(c, b)

                @pl.when(c >= 1)
                def _():
                    store_wait(c - 1, 1 - b)

                @pl.when(c + 1 < nchunk)
                def _():
                    gather_start(c + 1, 1 - b)

                store_start(c, b)
            return carry

        lax.fori_loop(0, nchunk // 2, body, 0)
        store_wait(nchunk - 1, (nchunk - 1) % 2)

    return sc_gather


_sc_gather_chunk = _make_sc_gather(BC)


ROWS = 2048
NSEQ = ROWS // SEQ
GRID_C = BC // ROWS
_FINV = 1.0 / DIM


def _tc_tail(g_ref, pos_ref, gam_ref, bet_ref, out_ref):
    u = g_ref[...].reshape(NSEQ, SEQ, HDIM)
    lo = lax.bitcast_convert_type(lax.shift_left(u, 16), jnp.float32)
    hi = lax.bitcast_convert_type(
        jnp.bitwise_and(u, jnp.int32(-65536)), jnp.float32)
    xl = lo + pos_ref[:, :HDIM][None]
    xh = hi + pos_ref[:, HDIM:][None]
    s = (jnp.sum(xl, axis=2, keepdims=True) + jnp.sum(xh, axis=2, keepdims=True))
    q = (jnp.sum(xl * xl, axis=2, keepdims=True)
         + jnp.sum(xh * xh, axis=2, keepdims=True))
    mean = s * _FINV
    var = q * _FINV - mean * mean
    r = lax.rsqrt(var + LN_EPS)
    yl = (xl - mean) * r * gam_ref[:, :HDIM][None] + bet_ref[:, :HDIM][None]
    yh = (xh - mean) * r * gam_ref[:, HDIM:][None] + bet_ref[:, HDIM:][None]
    out_ref[:, :HDIM] = yl.reshape(ROWS, HDIM)
    out_ref[:, HDIM:] = yh.reshape(ROWS, HDIM)


def _tc_tail_alias(prev_ref, g_ref, pos_ref, gam_ref, bet_ref, out_ref):
    del prev_ref
    _tc_tail(g_ref, pos_ref, gam_ref, bet_ref, out_ref)


def _chunk_specs():
    return [
        pl.BlockSpec((ROWS, HDIM), lambda i: (i, 0)),
        pl.BlockSpec((SEQ, DIM), lambda i: (0, 0)),
        pl.BlockSpec((1, DIM), lambda i: (0, 0)),
        pl.BlockSpec((1, DIM), lambda i: (0, 0)),
    ]


def _tc_chunk_call(k, prev, g_k, pos_c, gam, bet):
    out_map = functools.partial(lambda kk, i: (i + kk * GRID_C, 0), k)
    out_shape = jax.ShapeDtypeStruct((B, DIM), jnp.float32)
    if prev is None:
        return pl.pallas_call(
            _tc_tail,
            grid=(GRID_C,),
            in_specs=_chunk_specs(),
            out_specs=pl.BlockSpec((ROWS, DIM), out_map),
            out_shape=out_shape,
        )(g_k, pos_c, gam, bet)
    return pl.pallas_call(
        _tc_tail_alias,
        grid=(GRID_C,),
        in_specs=[pl.BlockSpec(memory_space=pl.ANY)] + _chunk_specs(),
        out_specs=pl.BlockSpec((ROWS, DIM), out_map),
        out_shape=out_shape,
        input_output_aliases={0: 0},
    )(prev, g_k, pos_c, gam, bet)


def kernel(input_ids, word_emb, extra_emb, token_type_emb, pos_emb, ln_gamma, ln_beta):
    ids = input_ids.reshape(-1).astype(jnp.int32)
    pos_c = pos_emb + token_type_emb[0][None, :]
    gam = ln_gamma.reshape(1, DIM)
    bet = ln_beta.reshape(1, DIM)

    table = _cast_table(word_emb, extra_emb)
    gathered = [
        _sc_gather_chunk(lax.slice(ids, (k * BC,), ((k + 1) * BC,)), table)
        for k in range(KCH)
    ]
    out = None
    for k in range(KCH):
        out = _tc_chunk_call(k, out, gathered[k], pos_c, gam, bet)
    return out.reshape(BATCH, SEQ, DIM)


# speedup vs baseline: 1.0637x; 1.0037x over previous
import functools

import jax
import jax.numpy as jnp
from jax import lax
from jax.experimental import pallas as pl
from jax.experimental.pallas import tpu as pltpu
from jax.experimental.pallas import tpu_sc as plsc

VOCAB = 30522
EXTRA = 2
DIM = 768
HDIM = DIM // 2
LN_EPS = 1e-12

BATCH = 128
SEQ = 512
B = BATCH * SEQ

NC = 2
NS = 16
NW = NC * NS
NB = 128

KCH = 4
BC = B // KCH

_sc_mesh = plsc.VectorSubcoreMesh(core_axis_name="c", subcore_axis_name="s")

CAST_ROWS = 1024
VTOT = VOCAB + EXTRA
_NCAST = pl.cdiv(VTOT, CAST_ROWS)
_E_BASE = VOCAB - (_NCAST - 1) * CAST_ROWS


def _pack_pair(x):
    lo = x[:, :HDIM].astype(jnp.bfloat16).astype(jnp.float32)
    hi = x[:, HDIM:].astype(jnp.bfloat16).astype(jnp.float32)
    lob = lax.bitcast_convert_type(lo, jnp.int32)
    hib = lax.bitcast_convert_type(hi, jnp.int32)
    return jnp.bitwise_or(hib, lax.shift_right_logical(lob, 16))


def _cast_body(x_ref, e_ref, o_ref):
    i = pl.program_id(0)
    packed = _pack_pair(x_ref[...])

    @pl.when(i == _NCAST - 1)
    def _():
        pe = _pack_pair(e_ref[...])
        row = lax.broadcasted_iota(jnp.int32, (CAST_ROWS, 1), 0)
        p = jnp.where(row == _E_BASE, pe[0:1, :], packed)
        o_ref[...] = jnp.where(row == _E_BASE + 1, pe[1:2, :], p)

    @pl.when(i != _NCAST - 1)
    def _():
        o_ref[...] = packed


def _cast_table(word_emb, extra_emb):
    return pl.pallas_call(
        _cast_body,
        grid=(_NCAST,),
        in_specs=[
            pl.BlockSpec((CAST_ROWS, DIM), lambda i: (i, 0)),
            pl.BlockSpec((EXTRA, DIM), lambda i: (0, 0)),
        ],
        out_specs=pl.BlockSpec((CAST_ROWS, HDIM), lambda i: (i, 0)),
        out_shape=jax.ShapeDtypeStruct((VTOT, HDIM), jnp.int32),
    )(word_emb, extra_emb)


def _make_sc_gather(bt):
    bpw = bt // NW
    nchunk = bpw // NB

    @functools.partial(
        pl.kernel,
        mesh=_sc_mesh,
        out_type=jax.ShapeDtypeStruct((bt, HDIM), jnp.int32),
        scratch_types=[
            pltpu.VMEM((bpw,), jnp.int32),
            pltpu.VMEM((NB, HDIM), jnp.int32),
            pltpu.VMEM((NB, HDIM), jnp.int32),
            pltpu.SemaphoreType.DMA,
            pltpu.SemaphoreType.DMA,
            pltpu.SemaphoreType.DMA,
            pltpu.SemaphoreType.DMA,
        ],
    )
    def sc_gather(ids_hbm, table_hbm, out_hbm, idx_v, rows0, rows1, g0, g1, s0, s1):
        wid = lax.axis_index("s") * NC + lax.axis_index("c")
        base = wid * bpw
        pltpu.sync_copy(ids_hbm.at[pl.ds(base, bpw)], idx_v)
        rows = (rows0, rows1)
        gsem = (g0, g1)
        ssem = (s0, s1)

        def gather_start(c, b):
            pltpu.async_copy(table_hbm.at[idx_v.at[pl.ds(c * NB, NB)]], rows[b], gsem[b])

        def gather_wait(c, b):
            pltpu.make_async_copy(
                table_hbm.at[idx_v.at[pl.ds(c * NB, NB)]], rows[b], gsem[b]
            ).wait()

        def store_start(c, b):
            pltpu.async_copy(rows[b], out_hbm.at[pl.ds(base + c * NB, NB)], ssem[b])

        def store_wait(c, b):
            pltpu.make_async_copy(
                rows[b], out_hbm.at[pl.ds(base + c * NB, NB)], ssem[b]
            ).wait()

        gather_start(0, 0)

        def body(i, carry):
            c0 = i * 2
            for b in range(2):
                c = c0 + b
                gather_wait(c, b)

                @pl.when(c >= 1)
                def _():
                    store_wait(c - 1, 1 - b)

                @pl.when(c + 1 < nchunk)
                def _():
                    gather_start(c + 1, 1 - b)

                store_start(c, b)
            return carry

        lax.fori_loop(0, nchunk // 2, body, 0)
        store_wait(nchunk - 1, (nchunk - 1) % 2)

    return sc_gather


_sc_gather_chunk = _make_sc_gather(BC)


ROWS = 4096
NSEQ = ROWS // SEQ
GRID_C = BC // ROWS
_FINV = 1.0 / DIM


def _tc_tail(g_ref, pos_ref, gam_ref, bet_ref, out_ref):
    u = g_ref[...].reshape(NSEQ, SEQ, HDIM)
    lo = lax.bitcast_convert_type(lax.shift_left(u, 16), jnp.float32)
    hi = lax.bitcast_convert_type(
        jnp.bitwise_and(u, jnp.int32(-65536)), jnp.float32)
    xl = lo + pos_ref[:, :HDIM][None]
    xh = hi + pos_ref[:, HDIM:][None]
    s = (jnp.sum(xl, axis=2, keepdims=True) + jnp.sum(xh, axis=2, keepdims=True))
    q = (jnp.sum(xl * xl, axis=2, keepdims=True)
         + jnp.sum(xh * xh, axis=2, keepdims=True))
    mean = s * _FINV
    var = q * _FINV - mean * mean
    r = lax.rsqrt(var + LN_EPS)
    yl = (xl - mean) * r * gam_ref[:, :HDIM][None] + bet_ref[:, :HDIM][None]
    yh = (xh - mean) * r * gam_ref[:, HDIM:][None] + bet_ref[:, HDIM:][None]
    out_ref[:, :HDIM] = yl.reshape(ROWS, HDIM)
    out_ref[:, HDIM:] = yh.reshape(ROWS, HDIM)


def _tc_tail_alias(prev_ref, g_ref, pos_ref, gam_ref, bet_ref, out_ref):
    del prev_ref
    _tc_tail(g_ref, pos_ref, gam_ref, bet_ref, out_ref)


def _chunk_specs():
    return [
        pl.BlockSpec((ROWS, HDIM), lambda i: (i, 0)),
        pl.BlockSpec((SEQ, DIM), lambda i: (0, 0)),
        pl.BlockSpec((1, DIM), lambda i: (0, 0)),
        pl.BlockSpec((1, DIM), lambda i: (0, 0)),
    ]


def _tc_chunk_call(k, prev, g_k, pos_c, gam, bet):
    out_map = functools.partial(lambda kk, i: (i + kk * GRID_C, 0), k)
    out_shape = jax.ShapeDtypeStruct((B, DIM), jnp.float32)
    if prev is None:
        return pl.pallas_call(
            _tc_tail,
            grid=(GRID_C,),
            in_specs=_chunk_specs(),
            out_specs=pl.BlockSpec((ROWS, DIM), out_map),
            out_shape=out_shape,
        )(g_k, pos_c, gam, bet)
    return pl.pallas_call(
        _tc_tail_alias,
        grid=(GRID_C,),
        in_specs=[pl.BlockSpec(memory_space=pl.ANY)] + _chunk_specs(),
        out_specs=pl.BlockSpec((ROWS, DIM), out_map),
        out_shape=out_shape,
        input_output_aliases={0: 0},
    )(prev, g_k, pos_c, gam, bet)


def kernel(input_ids, word_emb, extra_emb, token_type_emb, pos_emb, ln_gamma, ln_beta):
    ids = input_ids.reshape(-1).astype(jnp.int32)
    pos_c = pos_emb + token_type_emb[0][None, :]
    gam = ln_gamma.reshape(1, DIM)
    bet = ln_beta.reshape(1, DIM)

    table = _cast_table(word_emb, extra_emb)
    gathered = [
        _sc_gather_chunk(lax.slice(ids, (k * BC,), ((k + 1) * BC,)), table)
        for k in range(KCH)
    ]
    out = None
    for k in range(KCH):
        out = _tc_chunk_call(k, out, gathered[k], pos_c, gam, bet)
    return out.reshape(BATCH, SEQ, DIM)
